# baseline (speedup 1.0000x reference)
import functools

import jax
import jax.numpy as jnp
import numpy as np
from jax import lax
from jax.experimental import pallas as pl
from jax.experimental.pallas import tpu as pltpu
from jax.experimental.pallas import tpu_sc as plsc

N = 500000
NBATCH = 4
GX, GY, GZ = 352, 400, 10
NUM_CELLS = GX * GY * GZ
NKEY = NBATCH * NUM_CELLS
SENT = NKEY
NSEG = 240000
NSEG_P = 240128
ROWS_PER_TILE = NSEG_P // 16

LOW0, LOW1, LOW2 = np.float32(0.0), np.float32(-40.0), np.float32(-3.0)
VS0, VS1, VS2 = np.float32(0.2), np.float32(0.2), np.float32(0.4)

NW = 32
NRANGE = 64
RNG = NKEY // NRANGE
W_CELLS = NKEY // NW
CHUNK = 2000
NCHUNK = N // CHUNK
CCHUNK = 2000
NCC = W_CELLS // CCHUNK
STG = 4096

_MESH = plsc.VectorSubcoreMesh(core_axis_name="c", subcore_axis_name="s")
_SC_PARAMS = pltpu.CompilerParams(needs_layout_passes=False)


def _wid():
    return lax.axis_index("s") * 2 + lax.axis_index("c")


def _k1_body(xs, ys, zs, bi, keys, xb, yb, zb, bb, kb):
    w = _wid()

    def chunk(ch):
        off = pl.multiple_of(ch * CHUNK, 8)
        pltpu.sync_copy(xs.at[pl.ds(off, CHUNK)], xb)
        pltpu.sync_copy(ys.at[pl.ds(off, CHUNK)], yb)
        pltpu.sync_copy(zs.at[pl.ds(off, CHUNK)], zb)
        pltpu.sync_copy(bi.at[pl.ds(off, CHUNK)], bb)

        def vec(i, _):
            s = pl.ds(i * 16, 16)
            vx = ((xb[s] - LOW0) / VS0).astype(jnp.int32)
            vy = ((yb[s] - LOW1) / VS1).astype(jnp.int32)
            vz = ((zb[s] - LOW2) / VS2).astype(jnp.int32)
            inr = ((vx >= 0) & (vx < GX) & (vy >= 0) & (vy < GY)
                   & (vz >= 0) & (vz < GZ))
            key = bb[s] * NUM_CELLS + (vz * GY + vy) * GX + vx
            kb[s] = jnp.where(inr, key, SENT)
            return 0

        lax.fori_loop(0, CHUNK // 16, vec, 0)
        pltpu.sync_copy(kb, keys.at[pl.ds(off, CHUNK)])

    for c in range(8):
        ch = c * NW + w

        @pl.when(ch < NCHUNK)
        def _():
            chunk(ch)


def _k2a_body(keys, counts, occ, hist, kb, ob):
    w = _wid()
    ones = jnp.full((16,), 1, jnp.int32)

    for p in range(2):
        r = p * NW + w
        base = r * RNG

        def zero(i, _):
            hist[pl.ds(i * 16, 16)] = jnp.zeros((16,), jnp.int32)
            return 0

        lax.fori_loop(0, RNG // 16, zero, 0)

        def cb(t, _):
            pltpu.sync_copy(keys.at[pl.ds(pl.multiple_of(t * CHUNK, 8),
                                          CHUNK)], kb)

            def vb(i, _):
                idx = kb[pl.ds(i * 16, 16)] - base
                m = (idx >= 0) & (idx < RNG)
                plsc.addupdate_scatter(hist, [jnp.where(m, idx, 0)], ones,
                                       mask=m)
                return 0

            lax.fori_loop(0, CHUNK // 16, vb, 0)
            return 0

        lax.fori_loop(0, NCHUNK, cb, 0)

        def of(i, acc):
            c16 = hist[pl.ds(i * 16, 16)]
            return acc + jnp.where(c16 > 0, 1, 0).astype(jnp.int32)

        acc = lax.fori_loop(0, RNG // 16, of, jnp.zeros((16,), jnp.int32))
        ob[...] = acc
        pltpu.sync_copy(ob, occ.at[r])
        pltpu.sync_copy(hist, counts.at[pl.ds(pl.multiple_of(base, 8), RNG)])


def _k2b_body(counts, occ, ranks, uniq, cnt, tot, ob, cbuf, rb, idxs, cells,
              cnts, tb):
    w = _wid()
    lanes = lax.iota(jnp.int32, 16)
    pltpu.sync_copy(occ, ob)

    def pf(r, carry):
        run, p0 = carry
        t = jnp.sum(ob[r])
        p0 = jnp.where(r == 2 * w, run, p0)
        return (run + t, p0)

    total, pref0 = lax.fori_loop(0, NRANGE, pf, (0, 0))

    @pl.when(w == 0)
    def _():
        tb[...] = jnp.zeros((16,), jnp.int32) + total
        pltpu.sync_copy(tb, tot)

    def inits(i, _):
        idxs[pl.ds(i * 16, 16)] = NSEG + ((w * 16 + lanes + i) & 127)
        return 0

    lax.fori_loop(0, STG // 16, inits, 0)

    gbase = w * W_CELLS

    def cc(t, carry):
        rank_run, n = carry
        coff = pl.multiple_of(gbase + t * CCHUNK, 8)
        pltpu.sync_copy(counts.at[pl.ds(coff, CCHUNK)], cbuf)

        def vb(i, carry2):
            rank_run, n = carry2
            c16 = cbuf[pl.ds(i * 16, 16)]
            f = (c16 > 0).astype(jnp.int32)
            excl = plsc.cumsum(f) - f
            rank16 = excl + rank_run
            rb[pl.ds(i * 16, 16)] = rank16
            m = (c16 > 0) & (rank16 < NSEG)
            plsc.store_compressed(idxs.at[pl.ds(n, 16)], rank16, mask=m)
            plsc.store_compressed(cells.at[pl.ds(n, 16)],
                                  coff + i * 16 + lanes, mask=m)
            plsc.store_compressed(cnts.at[pl.ds(n, 16)],
                                  c16.astype(jnp.float32), mask=m)
            return (rank_run + jnp.sum(f),
                    n + jnp.sum(jnp.where(m, 1, 0).astype(jnp.int32)))

        rank_run, n = lax.fori_loop(0, CCHUNK // 16, vb, (rank_run, n))
        pltpu.sync_copy(rb, ranks.at[pl.ds(coff, CCHUNK)])
        flush = n >= STG // 2

        @pl.when(flush)
        def _():
            pltpu.sync_copy(cells, uniq.at[idxs])
            pltpu.sync_copy(cnts, cnt.at[idxs])

        return (rank_run, jnp.where(flush, 0, n))

    lax.fori_loop(0, NCC, cc, (pref0, 0))
    pltpu.sync_copy(cells, uniq.at[idxs])
    pltpu.sync_copy(cnts, cnt.at[idxs])


def _k3_body(keys, xs, ys, zs, it, ranks, psum,
             kb, gb, rkb, fxb, xb, yb, zb, ib, zbuf, s0, s1, s2, s3, sem):
    w = _wid()
    core = lax.axis_index("c")
    sub = lax.axis_index("s")
    lanes = lax.iota(jnp.int32, 16)
    tslice = pl.ds(sub * ROWS_PER_TILE, ROWS_PER_TILE)

    def zv(i, _):
        zbuf[pl.ds(i * 16, 16)] = jnp.zeros((16,), jnp.float32)
        return 0

    lax.fori_loop(0, ROWS_PER_TILE // 16, zv, 0)
    for sx in (s0, s1, s2, s3):
        pltpu.sync_copy(zbuf, sx.at[tslice])
    plsc.subcore_barrier()

    def chunk(ch):
        off = pl.multiple_of(ch * CHUNK, 8)
        pltpu.sync_copy(keys.at[pl.ds(off, CHUNK)], kb)

        def clampv(i, _):
            gb[pl.ds(i * 16, 16)] = jnp.minimum(kb[pl.ds(i * 16, 16)],
                                                NKEY - 1)
            return 0

        lax.fori_loop(0, CHUNK // 16, clampv, 0)
        pltpu.async_copy(ranks.at[gb], rkb, sem).wait()
        pltpu.sync_copy(xs.at[pl.ds(off, CHUNK)], xb)
        pltpu.sync_copy(ys.at[pl.ds(off, CHUNK)], yb)
        pltpu.sync_copy(zs.at[pl.ds(off, CHUNK)], zb)
        pltpu.sync_copy(it.at[pl.ds(off, CHUNK)], ib)

        def fixv(i, _):
            r16 = rkb[pl.ds(i * 16, 16)]
            k16 = kb[pl.ds(i * 16, 16)]
            valid = (k16 < SENT) & (r16 < NSEG)
            dump = NSEG + ((w * 16 + lanes + i) & 127)
            fxb[pl.ds(i * 16, 16)] = jnp.where(valid, r16, dump)
            return 0

        lax.fori_loop(0, CHUNK // 16, fixv, 0)
        pltpu.sync_copy(xb, s0.at[fxb], add=True)
        pltpu.sync_copy(yb, s1.at[fxb], add=True)
        pltpu.sync_copy(zb, s2.at[fxb], add=True)
        pltpu.sync_copy(ib, s3.at[fxb], add=True)

    for c in range(8):
        ch = c * NW + w

        @pl.when(ch < NCHUNK)
        def _():
            chunk(ch)

    plsc.subcore_barrier()
    for j, sx in enumerate((s0, s1, s2, s3)):
        po = pl.multiple_of((core * 4 + j) * NSEG_P + sub * ROWS_PER_TILE, 8)
        pltpu.sync_copy(sx.at[tslice], zbuf)
        pltpu.sync_copy(zbuf, psum.at[pl.ds(po, ROWS_PER_TILE)])


def _k4_body(tot, u, cn, a0, a1, a2, a3, b0, b1, b2, b3,
             e0, e1, e2, e3, cb, czb, cyb, cxb, vn):
    pid = pl.program_id(0)
    row = lax.broadcasted_iota(jnp.int32, (128, 128), 0)
    lane = lax.broadcasted_iota(jnp.int32, (128, 128), 1)
    idx = (pid * 128 + row) * 128 + lane
    valid = idx < tot[0]
    c = jnp.where(valid, cn[...], jnp.float32(0.0))
    norm = jnp.maximum(c, jnp.float32(1.0))
    vn[...] = c
    e0[...] = (a0[...] + b0[...]) / norm
    e1[...] = (a1[...] + b1[...]) / norm
    e2[...] = (a2[...] + b2[...]) / norm
    e3[...] = (a3[...] + b3[...]) / norm
    cell = jnp.where(valid, u[...], 0)
    b = lax.div(cell, jnp.int32(NUM_CELLS))
    rem = cell - b * NUM_CELLS
    vz = lax.div(rem, jnp.int32(GX * GY))
    r2 = rem - vz * (GX * GY)
    vy = lax.div(r2, jnp.int32(GX))
    vx = r2 - vy * GX
    cb[...] = b
    czb[...] = vz
    cyb[...] = vy
    cxb[...] = vx


_k1 = pl.kernel(
    _k1_body,
    out_type=jax.ShapeDtypeStruct((N,), jnp.int32),
    mesh=_MESH,
    compiler_params=_SC_PARAMS,
    scratch_types=[
        pltpu.VMEM((CHUNK,), jnp.float32),
        pltpu.VMEM((CHUNK,), jnp.float32),
        pltpu.VMEM((CHUNK,), jnp.float32),
        pltpu.VMEM((CHUNK,), jnp.int32),
        pltpu.VMEM((CHUNK,), jnp.int32),
    ],
)

_k2a = pl.kernel(
    _k2a_body,
    out_type=(
        jax.ShapeDtypeStruct((NKEY,), jnp.int32),
        jax.ShapeDtypeStruct((NRANGE, 16), jnp.int32),
    ),
    mesh=_MESH,
    compiler_params=_SC_PARAMS,
    scratch_types=[
        pltpu.VMEM((RNG,), jnp.int32),
        pltpu.VMEM((CHUNK,), jnp.int32),
        pltpu.VMEM((16,), jnp.int32),
    ],
)

_k2b = pl.kernel(
    _k2b_body,
    out_type=(
        jax.ShapeDtypeStruct((NKEY,), jnp.int32),
        jax.ShapeDtypeStruct((NSEG_P,), jnp.int32),
        jax.ShapeDtypeStruct((NSEG_P,), jnp.float32),
        jax.ShapeDtypeStruct((16,), jnp.int32),
    ),
    mesh=_MESH,
    compiler_params=_SC_PARAMS,
    scratch_types=[
        pltpu.VMEM((NRANGE, 16), jnp.int32),
        pltpu.VMEM((CCHUNK,), jnp.int32),
        pltpu.VMEM((CCHUNK,), jnp.int32),
        pltpu.VMEM((STG,), jnp.int32),
        pltpu.VMEM((STG,), jnp.int32),
        pltpu.VMEM((STG,), jnp.float32),
        pltpu.VMEM((16,), jnp.int32),
    ],
)

_k3 = pl.kernel(
    _k3_body,
    out_type=jax.ShapeDtypeStruct((8 * NSEG_P,), jnp.float32),
    mesh=_MESH,
    compiler_params=_SC_PARAMS,
    scratch_types=[
        pltpu.VMEM((CHUNK,), jnp.int32),
        pltpu.VMEM((CHUNK,), jnp.int32),
        pltpu.VMEM((CHUNK,), jnp.int32),
        pltpu.VMEM((CHUNK,), jnp.int32),
        pltpu.VMEM((CHUNK,), jnp.float32),
        pltpu.VMEM((CHUNK,), jnp.float32),
        pltpu.VMEM((CHUNK,), jnp.float32),
        pltpu.VMEM((CHUNK,), jnp.float32),
        pltpu.VMEM((ROWS_PER_TILE,), jnp.float32),
        pltpu.VMEM_SHARED((NSEG_P,), jnp.float32),
        pltpu.VMEM_SHARED((NSEG_P,), jnp.float32),
        pltpu.VMEM_SHARED((NSEG_P,), jnp.float32),
        pltpu.VMEM_SHARED((NSEG_P,), jnp.float32),
        pltpu.SemaphoreType.DMA,
    ],
)

NPAD = 245760


def _pad2d(x):
    return jnp.pad(x, (0, NPAD - NSEG)).reshape(1920, 128)


@functools.partial(jax.jit, static_argnums=(2,))
def _run(ldr_pc_64, pts_batch_indices, _unused=0):
    ldrT = jnp.swapaxes(ldr_pc_64, 0, 1)
    xs = ldrT[0]
    ys = ldrT[1]
    zs = ldrT[2]
    it = ldrT[3]
    keys = _k1(xs, ys, zs, pts_batch_indices)
    counts, occ = _k2a(keys)
    ranks, uniq, cnt, tot = _k2b(counts, occ)
    psum = _k3(keys, xs, ys, zs, it, ranks)

    totc = jnp.minimum(tot[:1], NSEG)
    u2 = _pad2d(uniq[:NSEG])
    cn2 = _pad2d(cnt[:NSEG])
    pcols = [_pad2d(psum[k * NSEG_P:k * NSEG_P + NSEG]) for k in range(8)]

    grid = 15
    bspec = pl.BlockSpec((128, 128), lambda i: (i, 0))
    outs = pl.pallas_call(
        _k4_body,
        grid=(grid,),
        in_specs=[pl.BlockSpec(memory_space=pltpu.SMEM)] + [bspec] * 10,
        out_specs=[bspec] * 9,
        out_shape=[jax.ShapeDtypeStruct((1920, 128), jnp.float32)] * 4
        + [jax.ShapeDtypeStruct((1920, 128), jnp.int32)] * 4
        + [jax.ShapeDtypeStruct((1920, 128), jnp.float32)],
    )(totc, u2, cn2, pcols[0], pcols[1], pcols[2], pcols[3],
      pcols[4], pcols[5], pcols[6], pcols[7])
    e = [o.reshape(-1)[:NSEG] for o in outs[:4]]
    co = [o.reshape(-1)[:NSEG] for o in outs[4:8]]
    vn = outs[8].reshape(-1)[:NSEG]
    encoded = jnp.stack(e, axis=1)
    coords = jnp.stack(co, axis=1)
    return encoded, coords, vn


def kernel(ldr_pc_64, pts_batch_indices, batch_size):
    del batch_size
    return _run(ldr_pc_64, pts_batch_indices)
